# baseline (speedup 1.0000x reference)
import functools

import jax
import jax.numpy as jnp
from jax import lax
from jax.experimental import pallas as pl
from jax.experimental.pallas import tpu as pltpu
from jax.experimental.pallas import tpu_sc as plsc

_CH = 128


@functools.lru_cache(maxsize=None)
def _build(S0, S1, V, D):
    info = plsc.get_sparse_core_info()
    nw = info.num_cores * info.num_subcores
    B = S0 * S1
    b_per_w = B // nw
    n_ch = b_per_w // _CH
    w_per_row = S1 // b_per_w
    assert b_per_w * nw == B and n_ch * _CH == b_per_w
    assert w_per_row * b_per_w == S1

    mesh = plsc.VectorSubcoreMesh(core_axis_name="c", subcore_axis_name="s")

    nbuf = 7
    depth = 5

    @functools.partial(
        pl.kernel,
        mesh=mesh,
        out_type=jax.ShapeDtypeStruct((S0, S1, D), jnp.float32),
        scratch_types=[
            pltpu.VMEM((b_per_w,), jnp.int32),
            pltpu.VMEM_SHARED((V, D), jnp.float32),
        ]
        + [pltpu.VMEM((_CH, D), jnp.float32) for _ in range(nbuf)]
        + [pltpu.SemaphoreType.DMA for _ in range(2 * nbuf)],
    )
    def gather_kernel(idx_hbm, table_hbm, out_hbm, idx_v, table_sh, *scratch):
        bufs = scratch[:nbuf]
        gsems = scratch[nbuf : 2 * nbuf]
        ssems = scratch[2 * nbuf :]
        sid = lax.axis_index("s")
        wid = sid * info.num_cores + lax.axis_index("c")
        row = wid // w_per_row
        col = (wid % w_per_row) * b_per_w
        @pl.when(sid == 0)
        def _():
            pltpu.sync_copy(table_hbm, table_sh)

        pltpu.sync_copy(idx_hbm.at[row, pl.ds(col, b_per_w)], idx_v)
        plsc.subcore_barrier()

        def gather(j):
            return pltpu.async_copy(
                table_sh.at[idx_v.at[pl.ds(j * _CH, _CH)]],
                bufs[j % nbuf],
                gsems[j % nbuf],
            )

        gathers = [None] * n_ch
        scatters = [None] * n_ch
        for j in range(min(depth, n_ch)):
            gathers[j] = gather(j)
        for j in range(n_ch):
            gathers[j].wait()
            scatters[j] = pltpu.async_copy(
                bufs[j % nbuf],
                out_hbm.at[row, pl.ds(col + j * _CH, _CH), :],
                ssems[j % nbuf],
            )
            k = j + depth
            if k < n_ch:
                if k >= nbuf:
                    scatters[k - nbuf].wait()
                    scatters[k - nbuf] = None
                gathers[k] = gather(k)
        for j in range(n_ch):
            if scatters[j] is not None:
                scatters[j].wait()

    return gather_kernel


def kernel(pinyin_ids, table):
    s0, s1 = pinyin_ids.shape
    V, D = table.shape
    return _build(s0, s1, V, D)(pinyin_ids, table)


# speedup vs baseline: 1.0060x; 1.0060x over previous
import functools

import jax
import jax.numpy as jnp
from jax import lax
from jax.experimental import pallas as pl
from jax.experimental.pallas import tpu as pltpu
from jax.experimental.pallas import tpu_sc as plsc

_CH = 64


@functools.lru_cache(maxsize=None)
def _build(S0, S1, V, D):
    info = plsc.get_sparse_core_info()
    nw = info.num_cores * info.num_subcores
    B = S0 * S1
    b_per_w = B // nw
    n_ch = b_per_w // _CH
    w_per_row = S1 // b_per_w
    assert b_per_w * nw == B and n_ch * _CH == b_per_w
    assert w_per_row * b_per_w == S1

    mesh = plsc.VectorSubcoreMesh(core_axis_name="c", subcore_axis_name="s")

    nbuf = 12
    depth = 8

    @functools.partial(
        pl.kernel,
        mesh=mesh,
        out_type=jax.ShapeDtypeStruct((S0, S1, D), jnp.float32),
        scratch_types=[
            pltpu.VMEM((b_per_w,), jnp.int32),
            pltpu.VMEM_SHARED((V, D), jnp.float32),
        ]
        + [pltpu.VMEM((_CH, D), jnp.float32) for _ in range(nbuf)]
        + [pltpu.SemaphoreType.DMA for _ in range(2 * nbuf)],
    )
    def gather_kernel(idx_hbm, table_hbm, out_hbm, idx_v, table_sh, *scratch):
        bufs = scratch[:nbuf]
        gsems = scratch[nbuf : 2 * nbuf]
        ssems = scratch[2 * nbuf :]
        sid = lax.axis_index("s")
        wid = sid * info.num_cores + lax.axis_index("c")
        row = wid // w_per_row
        col = (wid % w_per_row) * b_per_w
        @pl.when(sid == 0)
        def _():
            pltpu.sync_copy(table_hbm, table_sh)

        pltpu.sync_copy(idx_hbm.at[row, pl.ds(col, b_per_w)], idx_v)
        plsc.subcore_barrier()

        def gather(j):
            return pltpu.async_copy(
                table_sh.at[idx_v.at[pl.ds(j * _CH, _CH)]],
                bufs[j % nbuf],
                gsems[j % nbuf],
            )

        gathers = [None] * n_ch
        scatters = [None] * n_ch
        for j in range(min(depth, n_ch)):
            gathers[j] = gather(j)
        for j in range(n_ch):
            gathers[j].wait()
            scatters[j] = pltpu.async_copy(
                bufs[j % nbuf],
                out_hbm.at[row, pl.ds(col + j * _CH, _CH), :],
                ssems[j % nbuf],
            )
            k = j + depth
            if k < n_ch:
                if k >= nbuf:
                    scatters[k - nbuf].wait()
                    scatters[k - nbuf] = None
                gathers[k] = gather(k)
        for j in range(n_ch):
            if scatters[j] is not None:
                scatters[j].wait()

    return gather_kernel


def kernel(pinyin_ids, table):
    s0, s1 = pinyin_ids.shape
    V, D = table.shape
    return _build(s0, s1, V, D)(pinyin_ids, table)
